# Initial kernel scaffold; baseline (speedup 1.0000x reference)
import jax
import jax.numpy as jnp
from jax.experimental import pallas as pl


def kernel(msa, pair, xyz, state, idx, top_k, params):
    raise NotImplementedError("write your pallas kernel here")


# speedup vs baseline: 3.7718x; 3.7718x over previous
import functools

import jax
import jax.numpy as jnp
from jax import lax
from jax.experimental import pallas as pl
from jax.experimental.pallas import tpu as pltpu

L = 512
TOPK = 128
BI = 8
NBLK = L // BI
D_MSA, D_PAIR, D_STATE = 256, 128, 16
L0_IN, L0_OUT, D_EDGE = 32, 16, 32
HID = 64


def _ln(x, g, b, eps=1e-5):
    mu = jnp.mean(x, axis=-1, keepdims=True)
    var = jnp.mean((x - mu) ** 2, axis=-1, keepdims=True)
    return (x - mu) * lax.rsqrt(var + eps) * g + b


def _dot(a, b):
    return jnp.dot(a, b, preferred_element_type=jnp.float32)


def _prep_body(msa_ref, st_ref, xr_ref, xT_ref,
               gmsa_ref, bmsa_ref, gst_ref, bst_ref,
               Wx_ref, bx_ref, gn_ref, bn_ref, W1_ref,
               h_ref, sW_ref, dW_ref, D_ref, sel_ref, l1_ref):
    m = _ln(msa_ref[...], gmsa_ref[...], bmsa_ref[...])
    st = _ln(st_ref[...], gst_ref[...], bst_ref[...])
    x = jnp.concatenate([m, st], axis=1)
    h = _ln(_dot(x, Wx_ref[...]) + bx_ref[...], gn_ref[...], bn_ref[...])
    h_ref[...] = h
    W1 = W1_ref[...]
    sW_ref[...] = _dot(h, W1[0:L0_IN, :])
    dW_ref[...] = _dot(h, W1[L0_IN:2 * L0_IN, :])

    xr = xr_ref[...]
    ca = xr[:, 3:6]
    l1_ref[...] = xr - jnp.concatenate([ca, ca, ca], axis=1)

    d2 = jnp.zeros((L, L), jnp.float32)
    for x3 in range(3):
        col = xr[:, 3 + x3:4 + x3]
        row = xT_ref[3 + x3:4 + x3, :]
        dx = col - row
        d2 = d2 + dx * dx
    D = jnp.sqrt(d2 + 1e-8)
    D_ref[...] = D

    ri = lax.broadcasted_iota(jnp.int32, (L, L), 0)
    ci = lax.broadcasted_iota(jnp.int32, (L, L), 1)
    Dg = D + jnp.where(ri == ci, jnp.float32(999.9), jnp.float32(0.0))
    bits = lax.bitcast_convert_type(Dg, jnp.int32)

    lo0 = jnp.zeros((L, 1), jnp.int32)
    hi0 = jnp.full((L, 1), jnp.int32(0x7F7FFFFF))

    def body(_, carry):
        lo, hi = carry
        mid = lo + (hi - lo) // 2
        cnt = jnp.sum((bits <= mid).astype(jnp.int32), axis=1, keepdims=True)
        ge = cnt >= TOPK
        return jnp.where(ge, lo, mid + 1), jnp.where(ge, mid, hi)

    lo, hi = lax.fori_loop(0, 31, body, (lo0, hi0))
    t = hi
    below = bits < t
    ties = bits == t
    cnt_lt = jnp.sum(below.astype(jnp.float32), axis=1, keepdims=True)
    need = jnp.float32(TOPK) - cnt_lt
    ties_f = ties.astype(jnp.float32)
    upper = (ri <= ci).astype(jnp.float32)
    rank = _dot(ties_f, upper)
    sel = jnp.where(below | (ties & (rank <= need)),
                    jnp.float32(1.0), jnp.float32(0.0))
    sel_ref[...] = sel


def _prep(msa0, state0, xr, xT, p):
    out_shapes = [
        jax.ShapeDtypeStruct((L, L0_IN), jnp.float32),
        jax.ShapeDtypeStruct((L, HID), jnp.float32),
        jax.ShapeDtypeStruct((L, HID), jnp.float32),
        jax.ShapeDtypeStruct((L, L), jnp.float32),
        jax.ShapeDtypeStruct((L, L), jnp.float32),
        jax.ShapeDtypeStruct((L, 9), jnp.float32),
    ]
    r2 = lambda v: v.reshape(1, -1)
    args = (msa0, state0, xr, xT,
            r2(p['ln_msa'][0]), r2(p['ln_msa'][1]),
            r2(p['ln_state'][0]), r2(p['ln_state'][1]),
            p['embed_x'][0], r2(p['embed_x'][1]),
            r2(p['ln_node'][0]), r2(p['ln_node'][1]),
            p['se3_W1'][0])
    return pl.pallas_call(
        _prep_body,
        out_shape=out_shapes,
    )(*args)


def _edge_body(pair_ref, Dc_ref, selT_ref, xrb_ref, xr_ref, idxc_ref, idxb_ref,
               sWb_ref, dW_ref, l1b_ref,
               gp_ref, bp_ref,
               We1_ref, be1_ref, ge1_ref, bn1_ref,
               We2a_ref, We2r_ref, we2s_ref, be2_ref, ge2_ref, bn2_ref,
               W1pr_ref, w1d_ref, b1_ref, W2_ref, b2_ref,
               aggs_ref, aggv_ref):
    @pl.when(pl.program_id(0) == 0)
    def _():
        aggs_ref[...] = jnp.zeros_like(aggs_ref)
        aggv_ref[...] = jnp.zeros_like(aggv_ref)

    mu = (lax.broadcasted_iota(jnp.int32, (1, 36), 1).astype(jnp.float32)
          * jnp.float32(20.0 / 35.0))
    sig = jnp.float32(20.0 / 36)
    dW = dW_ref[...]
    idxc = idxc_ref[...]
    aggs_acc = jnp.zeros_like(aggs_ref)
    aggv_cols = [jnp.zeros((L, 1), jnp.float32) for _ in range(6)]
    for r in range(BI):
        pairf = _ln(pair_ref[r], gp_ref[...], bp_ref[...])
        pr = _ln(_dot(pairf, We1_ref[...]) + be1_ref[...],
                 ge1_ref[...], bn1_ref[...])
        Dcol = Dc_ref[0][:, r:r + 1]
        rbf = jnp.exp(-(((Dcol - mu) / sig) ** 2))
        dseq = idxc - idxb_ref[r:r + 1, 0:1]
        aseq = jnp.abs(dseq)
        seq = jnp.sign(dseq) * jnp.where(aseq > 1.0, 0.0, aseq)
        pr2 = (_dot(pr, We2a_ref[...]) + _dot(rbf, We2r_ref[...])
               + seq * we2s_ref[...] + be2_ref[...])
        pr2 = _ln(pr2, ge2_ref[...], bn2_ref[...])
        hid = (_dot(pr2, W1pr_ref[...]) + Dcol * w1d_ref[...] + b1_ref[...]
               + sWb_ref[r:r + 1, :] + dW)
        hid = jnp.maximum(hid, 0.0)
        out = _dot(hid, W2_ref[...]) + b2_ref[...]
        mcol = selT_ref[0][:, r:r + 1]
        aggs_acc = aggs_acc + out[:, :L0_IN] * mcol
        for c in range(2):
            for x3 in range(3):
                u = ((xr_ref[...][:, 3 + x3:4 + x3]
                      - xrb_ref[r:r + 1, 3 + x3:4 + x3]) / (Dcol + 1e-6))
                mv = out[:, L0_IN + 4 * c:L0_IN + 4 * c + 1] * u
                for b in range(3):
                    mv = mv + (out[:, L0_IN + 4 * c + 1 + b:L0_IN + 4 * c + 2 + b]
                               * l1b_ref[r:r + 1, 3 * b + x3:3 * b + x3 + 1])
                aggv_cols[3 * c + x3] = aggv_cols[3 * c + x3] + mv * mcol
    aggs_ref[...] += aggs_acc
    aggv_ref[...] += jnp.concatenate(aggv_cols, axis=1)


def _edges(pair3, D, selT, xr, idxc, sW, dW, l1, p):
    We2 = p['embed_e2'][0]
    W1 = p['se3_W1'][0]
    r2 = lambda v: v.reshape(1, -1)
    args = (pair3, D, selT, xr, xr, idxc, idxc, sW, dW, l1,
            r2(p['ln_pair'][0]), r2(p['ln_pair'][1]),
            p['embed_e1'][0], r2(p['embed_e1'][1]),
            r2(p['ln_edge1'][0]), r2(p['ln_edge1'][1]),
            We2[0:D_EDGE, :], We2[D_EDGE:D_EDGE + 36, :], We2[D_EDGE + 36:, :],
            r2(p['embed_e2'][1]),
            r2(p['ln_edge2'][0]), r2(p['ln_edge2'][1]),
            W1[2 * L0_IN:2 * L0_IN + D_EDGE, :], W1[2 * L0_IN + D_EDGE:, :],
            r2(p['se3_W1'][1]),
            p['se3_W2'][0], r2(p['se3_W2'][1]))
    in_specs = [
        pl.BlockSpec((BI, L, D_PAIR), lambda i: (i, 0, 0)),
        pl.BlockSpec((1, L, BI), lambda i: (i, 0, 0)),
        pl.BlockSpec((1, L, BI), lambda i: (i, 0, 0)),
        pl.BlockSpec((BI, 9), lambda i: (i, 0)),
        pl.BlockSpec((L, 9), lambda i: (0, 0)),
        pl.BlockSpec((L, 1), lambda i: (0, 0)),
        pl.BlockSpec((BI, 1), lambda i: (i, 0)),
        pl.BlockSpec((BI, HID), lambda i: (i, 0)),
        pl.BlockSpec((L, HID), lambda i: (0, 0)),
        pl.BlockSpec((BI, 9), lambda i: (i, 0)),
    ] + [pl.BlockSpec(a.shape, functools.partial(lambda n, i: (0,) * n, a.ndim))
         for a in args[10:]]
    out_specs = [
        pl.BlockSpec((L, L0_IN), lambda i: (0, 0)),
        pl.BlockSpec((L, 6), lambda i: (0, 0)),
    ]
    out_shapes = [
        jax.ShapeDtypeStruct((L, L0_IN), jnp.float32),
        jax.ShapeDtypeStruct((L, 6), jnp.float32),
    ]
    return pl.pallas_call(
        _edge_body,
        grid=(NBLK,),
        in_specs=in_specs,
        out_specs=out_specs,
        out_shape=out_shapes,
        compiler_params=pltpu.CompilerParams(
            dimension_semantics=("arbitrary",)),
    )(*args)


def _finish_body(h_ref, aggs_ref, aggv_ref, l1_ref, xr_ref,
                 W0_ref, b0_ref, st_ref, xyz_ref, eps=1e-5):
    W0 = W0_ref[...]
    st_ref[...] = (_dot(h_ref[...], W0[0:L0_IN, :])
                   + _dot(aggs_ref[...], W0[L0_IN:, :]) + b0_ref[...])
    aggv = aggv_ref[...] * jnp.float32(0.01)
    T = aggv[:, 0:3]
    R = aggv[:, 3:6]
    Ra = jnp.sqrt(jnp.sum(R * R, axis=1, keepdims=True) + 1e-12)
    Rv = R / (Ra + eps)
    cosA = jnp.cos(Ra)
    sinA = jnp.sin(Ra)
    ca = xr_ref[...][:, 3:6]
    l1 = l1_ref[...]
    shift = ca + T
    outs = []
    for b in range(3):
        v = l1[:, 3 * b:3 * b + 3]
        Rdv = jnp.sum(Rv * v, axis=1, keepdims=True)
        cr = jnp.concatenate([
            Rv[:, 1:2] * v[:, 2:3] - Rv[:, 2:3] * v[:, 1:2],
            Rv[:, 2:3] * v[:, 0:1] - Rv[:, 0:1] * v[:, 2:3],
            Rv[:, 0:1] * v[:, 1:2] - Rv[:, 1:2] * v[:, 0:1],
        ], axis=1)
        u_par = Rv * Rdv
        vnew = (v - u_par) * cosA + cr * sinA + u_par
        outs.append(vnew + shift)
    xyz_ref[...] = jnp.concatenate(outs, axis=1)


def _finish(h, aggs, aggv, l1, xr, p):
    out_shapes = [
        jax.ShapeDtypeStruct((L, L0_OUT), jnp.float32),
        jax.ShapeDtypeStruct((L, 9), jnp.float32),
    ]
    return pl.pallas_call(
        _finish_body,
        out_shape=out_shapes,
    )(h, aggs, aggv, l1, xr, p['se3_W0'][0], p['se3_W0'][1].reshape(1, -1))


def kernel(msa, pair, xyz, state, idx, top_k, params):
    del top_k
    msa0 = msa[0, 0]
    state0 = state[0]
    xr = xyz[0].reshape(L, 9).astype(jnp.float32)
    xT = xr.T
    pair3 = pair[0]
    idxc = idx[0].astype(jnp.float32).reshape(L, 1)

    h, sW, dW, D, sel, l1 = _prep(msa0, state0, xr, xT, params)
    Dr = D.reshape(NBLK, BI, L).transpose(0, 2, 1)
    selTr = sel.reshape(NBLK, BI, L).transpose(0, 2, 1)
    aggs, aggv = _edges(pair3, Dr, selTr, xr, idxc, sW, dW, l1, params)
    st, xyz9 = _finish(h, aggs, aggv, l1, xr, params)
    return xyz9.reshape(1, L, 3, 3), st.reshape(1, L, L0_OUT)


# speedup vs baseline: 9.9708x; 2.6435x over previous
import functools

import jax
import jax.numpy as jnp
from jax import lax
from jax.experimental import pallas as pl
from jax.experimental.pallas import tpu as pltpu

L = 512
TOPK = 128
BI = 8
NBLK = L // BI
D_MSA, D_PAIR, D_STATE = 256, 128, 16
L0_IN, L0_OUT, D_EDGE = 32, 16, 32
HID = 64


def _ln(x, g, b, eps=1e-5):
    mu = jnp.mean(x, axis=-1, keepdims=True)
    var = jnp.mean((x - mu) ** 2, axis=-1, keepdims=True)
    return (x - mu) * lax.rsqrt(var + eps) * g + b


def _dot(a, b):
    return jnp.dot(a, b, preferred_element_type=jnp.float32)


def _prep_body(msa_ref, st_ref, xr_ref, xT_ref,
               gmsa_ref, bmsa_ref, gst_ref, bst_ref,
               Wx_ref, bx_ref, gn_ref, bn_ref, W1_ref,
               h_ref, sW_ref, dW_ref, D_ref, sel_ref, l1_ref):
    m = _ln(msa_ref[...], gmsa_ref[...], bmsa_ref[...])
    st = _ln(st_ref[...], gst_ref[...], bst_ref[...])
    x = jnp.concatenate([m, st], axis=1)
    h = _ln(_dot(x, Wx_ref[...]) + bx_ref[...], gn_ref[...], bn_ref[...])
    h_ref[...] = h
    W1 = W1_ref[...]
    sW_ref[...] = _dot(h, W1[0:L0_IN, :])
    dW_ref[...] = _dot(h, W1[L0_IN:2 * L0_IN, :])

    xr = xr_ref[...]
    ca = xr[:, 3:6]
    l1_ref[...] = xr - jnp.concatenate([ca, ca, ca], axis=1)

    d2 = jnp.zeros((L, L), jnp.float32)
    for x3 in range(3):
        col = xr[:, 3 + x3:4 + x3]
        row = xT_ref[3 + x3:4 + x3, :]
        dx = col - row
        d2 = d2 + dx * dx
    D = jnp.sqrt(d2 + 1e-8)
    D_ref[...] = D

    ri = lax.broadcasted_iota(jnp.int32, (L, L), 0)
    ci = lax.broadcasted_iota(jnp.int32, (L, L), 1)
    Dg = D + jnp.where(ri == ci, jnp.float32(999.9), jnp.float32(0.0))
    bits = lax.bitcast_convert_type(Dg, jnp.int32)

    lo0 = jnp.zeros((L, 1), jnp.int32)
    hi0 = jnp.full((L, 1), jnp.int32(0x7F7FFFFF))

    def body(_, carry):
        lo, hi = carry
        mid = lo + (hi - lo) // 2
        cnt = jnp.sum((bits <= mid).astype(jnp.int32), axis=1, keepdims=True)
        ge = cnt >= TOPK
        return jnp.where(ge, lo, mid + 1), jnp.where(ge, mid, hi)

    lo, hi = lax.fori_loop(0, 31, body, (lo0, hi0))
    t = hi
    below = bits < t
    ties = bits == t
    cnt_lt = jnp.sum(below.astype(jnp.float32), axis=1, keepdims=True)
    need = jnp.float32(TOPK) - cnt_lt
    ties_f = ties.astype(jnp.float32)
    upper = (ri <= ci).astype(jnp.float32)
    rank = _dot(ties_f, upper)
    sel = jnp.where(below | (ties & (rank <= need)),
                    jnp.float32(1.0), jnp.float32(0.0))
    sel_ref[...] = sel


def _prep(msa0, state0, xr, xT, p):
    out_shapes = [
        jax.ShapeDtypeStruct((L, L0_IN), jnp.float32),
        jax.ShapeDtypeStruct((L, HID), jnp.float32),
        jax.ShapeDtypeStruct((L, HID), jnp.float32),
        jax.ShapeDtypeStruct((L, L), jnp.float32),
        jax.ShapeDtypeStruct((L, L), jnp.float32),
        jax.ShapeDtypeStruct((L, 9), jnp.float32),
    ]
    r2 = lambda v: v.reshape(1, -1)
    args = (msa0, state0, xr, xT,
            r2(p['ln_msa'][0]), r2(p['ln_msa'][1]),
            r2(p['ln_state'][0]), r2(p['ln_state'][1]),
            p['embed_x'][0], r2(p['embed_x'][1]),
            r2(p['ln_node'][0]), r2(p['ln_node'][1]),
            p['se3_W1'][0])
    return pl.pallas_call(
        _prep_body,
        out_shape=out_shapes,
    )(*args)


def _edge_body(pair_ref, Dc_ref, selT_ref, xrb_ref, xr_ref, idxc_ref, idxb_ref,
               sWb_ref, dW_ref, l13_ref,
               Wg1_ref, cg1_ref, cb1_ref, o128_ref,
               W2ag_ref, cg2_ref, We2r_ref, we2s_ref, cb2_ref, o32_ref,
               W1g_ref, cg3_ref, w1d_ref, cb3_ref, o64_ref, W2_ref, b2_ref,
               aggs_ref, aggv_ref):
    @pl.when(pl.program_id(0) == 0)
    def _():
        aggs_ref[...] = jnp.zeros_like(aggs_ref)
        aggv_ref[...] = jnp.zeros_like(aggv_ref)

    mu = (lax.broadcasted_iota(jnp.int32, (1, 36), 1).astype(jnp.float32)
          * jnp.float32(20.0 / 35.0))
    sig = jnp.float32(20.0 / 36)
    eps = jnp.float32(1e-5)
    dW = dW_ref[...]
    ca_d = xr_ref[...][:, 3:6]
    Dc = Dc_ref[0]
    selT = selT_ref[0]
    dseq = idxc_ref[...] - idxb_ref[0]
    aseq = jnp.abs(dseq)
    seq8 = jnp.sign(dseq) * jnp.where(aseq > 1.0, 0.0, aseq)
    aggs_acc = jnp.zeros_like(aggs_ref)
    aggv_acc = jnp.zeros_like(aggv_ref)
    for r in range(BI):
        x0 = pair_ref[r]
        mu0 = _dot(x0, o128_ref[...])
        m20 = _dot(x0 * x0, o128_ref[...])
        rs0 = lax.rsqrt(m20 - mu0 * mu0 + eps)
        pre1 = rs0 * (_dot(x0, Wg1_ref[...]) - mu0 * cg1_ref[...]) + cb1_ref[...]
        mu1 = _dot(pre1, o32_ref[...])
        m21 = _dot(pre1 * pre1, o32_ref[...])
        rs1 = lax.rsqrt(m21 - mu1 * mu1 + eps)
        Dcol = Dc[:, r:r + 1]
        rbf = jnp.exp(-(((Dcol - mu) / sig) ** 2))
        pre2 = (rs1 * (_dot(pre1, W2ag_ref[...]) - mu1 * cg2_ref[...])
                + _dot(rbf, We2r_ref[...])
                + seq8[:, r:r + 1] * we2s_ref[...] + cb2_ref[...])
        mu2 = _dot(pre2, o64_ref[...])
        m22 = _dot(pre2 * pre2, o64_ref[...])
        rs2 = lax.rsqrt(m22 - mu2 * mu2 + eps)
        hid = (rs2 * (_dot(pre2, W1g_ref[...]) - mu2 * cg3_ref[...])
               + Dcol * w1d_ref[...] + cb3_ref[...]
               + sWb_ref[r:r + 1, :] + dW)
        hid = jnp.maximum(hid, 0.0)
        out = _dot(hid, W2_ref[...]) + b2_ref[...]
        mcol = selT[:, r:r + 1]
        aggs_acc = aggs_acc + out[:, :L0_IN] * mcol
        u3 = (ca_d - xrb_ref[r:r + 1, 3:6]) / (Dcol + 1e-6)
        l1m = l13_ref[r]
        mv0 = out[:, L0_IN:L0_IN + 1] * u3 + _dot(out[:, L0_IN + 1:L0_IN + 4], l1m)
        mv1 = out[:, L0_IN + 4:L0_IN + 5] * u3 + _dot(out[:, L0_IN + 5:L0_IN + 8], l1m)
        aggv_acc = aggv_acc + jnp.concatenate([mv0, mv1], axis=1) * mcol
    aggs_ref[...] += aggs_acc
    aggv_ref[...] += aggv_acc


def _edges(pair3, D, selT, xr, idxc, idxr3, sW, dW, l13, p):
    We2 = p['embed_e2'][0]
    W1 = p['se3_W1'][0]
    r2 = lambda v: v.reshape(1, -1)
    gp, bp = p['ln_pair']
    ge1, bn1 = p['ln_edge1']
    ge2, bn2 = p['ln_edge2']
    We1, be1 = p['embed_e1']
    We2a = We2[0:D_EDGE, :]
    We2r = We2[D_EDGE:D_EDGE + 36, :]
    we2s = We2[D_EDGE + 36:, :]
    W1pr = W1[2 * L0_IN:2 * L0_IN + D_EDGE, :]
    w1d = W1[2 * L0_IN + D_EDGE:, :]
    Wg1 = We1 * gp[:, None]
    cg1 = r2(jnp.sum(Wg1, axis=0))
    cb1 = r2(bp @ We1 + be1)
    W2ag = We2a * ge1[:, None]
    cg2 = r2(jnp.sum(W2ag, axis=0))
    cb2 = r2(bn1 @ We2a + p['embed_e2'][1])
    W1g = W1pr * ge2[:, None]
    cg3 = r2(jnp.sum(W1g, axis=0))
    cb3 = r2(bn2 @ W1pr + p['se3_W1'][1])
    o128 = jnp.full((D_PAIR, D_EDGE), 1.0 / D_PAIR, jnp.float32)
    o32 = jnp.full((D_EDGE, D_EDGE), 1.0 / D_EDGE, jnp.float32)
    o64 = jnp.full((D_EDGE, HID), 1.0 / D_EDGE, jnp.float32)
    args = (pair3, D, selT, xr, xr, idxc, idxr3, sW, dW, l13,
            Wg1, cg1, cb1, o128,
            W2ag, cg2, We2r, we2s, cb2, o32,
            W1g, cg3, w1d, cb3, o64,
            p['se3_W2'][0], r2(p['se3_W2'][1]))
    in_specs = [
        pl.BlockSpec((BI, L, D_PAIR), lambda i: (i, 0, 0)),
        pl.BlockSpec((1, L, BI), lambda i: (i, 0, 0)),
        pl.BlockSpec((1, L, BI), lambda i: (i, 0, 0)),
        pl.BlockSpec((BI, 9), lambda i: (i, 0)),
        pl.BlockSpec((L, 9), lambda i: (0, 0)),
        pl.BlockSpec((L, 1), lambda i: (0, 0)),
        pl.BlockSpec((1, 1, BI), lambda i: (i, 0, 0)),
        pl.BlockSpec((BI, HID), lambda i: (i, 0)),
        pl.BlockSpec((L, HID), lambda i: (0, 0)),
        pl.BlockSpec((BI, 3, 3), lambda i: (i, 0, 0)),
    ] + [pl.BlockSpec(a.shape, functools.partial(lambda n, i: (0,) * n, a.ndim))
         for a in args[10:]]
    out_specs = [
        pl.BlockSpec((L, L0_IN), lambda i: (0, 0)),
        pl.BlockSpec((L, 6), lambda i: (0, 0)),
    ]
    out_shapes = [
        jax.ShapeDtypeStruct((L, L0_IN), jnp.float32),
        jax.ShapeDtypeStruct((L, 6), jnp.float32),
    ]
    return pl.pallas_call(
        _edge_body,
        grid=(NBLK,),
        in_specs=in_specs,
        out_specs=out_specs,
        out_shape=out_shapes,
        compiler_params=pltpu.CompilerParams(
            dimension_semantics=("arbitrary",)),
    )(*args)


def _finish_body(h_ref, aggs_ref, aggv_ref, l1_ref, xr_ref,
                 W0_ref, b0_ref, st_ref, xyz_ref, eps=1e-5):
    W0 = W0_ref[...]
    st_ref[...] = (_dot(h_ref[...], W0[0:L0_IN, :])
                   + _dot(aggs_ref[...], W0[L0_IN:, :]) + b0_ref[...])
    aggv = aggv_ref[...] * jnp.float32(0.01)
    T = aggv[:, 0:3]
    R = aggv[:, 3:6]
    Ra = jnp.sqrt(jnp.sum(R * R, axis=1, keepdims=True) + 1e-12)
    Rv = R / (Ra + eps)
    cosA = jnp.cos(Ra)
    sinA = jnp.sin(Ra)
    ca = xr_ref[...][:, 3:6]
    l1 = l1_ref[...]
    shift = ca + T
    outs = []
    for b in range(3):
        v = l1[:, 3 * b:3 * b + 3]
        Rdv = jnp.sum(Rv * v, axis=1, keepdims=True)
        cr = jnp.concatenate([
            Rv[:, 1:2] * v[:, 2:3] - Rv[:, 2:3] * v[:, 1:2],
            Rv[:, 2:3] * v[:, 0:1] - Rv[:, 0:1] * v[:, 2:3],
            Rv[:, 0:1] * v[:, 1:2] - Rv[:, 1:2] * v[:, 0:1],
        ], axis=1)
        u_par = Rv * Rdv
        vnew = (v - u_par) * cosA + cr * sinA + u_par
        outs.append(vnew + shift)
    xyz_ref[...] = jnp.concatenate(outs, axis=1)


def _finish(h, aggs, aggv, l1, xr, p):
    out_shapes = [
        jax.ShapeDtypeStruct((L, L0_OUT), jnp.float32),
        jax.ShapeDtypeStruct((L, 9), jnp.float32),
    ]
    return pl.pallas_call(
        _finish_body,
        out_shape=out_shapes,
    )(h, aggs, aggv, l1, xr, p['se3_W0'][0], p['se3_W0'][1].reshape(1, -1))


def kernel(msa, pair, xyz, state, idx, top_k, params):
    del top_k
    msa0 = msa[0, 0]
    state0 = state[0]
    xr = xyz[0].reshape(L, 9).astype(jnp.float32)
    xT = xr.T
    pair3 = pair[0]
    idxc = idx[0].astype(jnp.float32).reshape(L, 1)

    h, sW, dW, D, sel, l1 = _prep(msa0, state0, xr, xT, params)
    Dr = D.reshape(NBLK, BI, L).transpose(0, 2, 1)
    selTr = sel.reshape(NBLK, BI, L).transpose(0, 2, 1)
    idxr3 = idxc.reshape(NBLK, 1, BI)
    l13 = l1.reshape(L, 3, 3)
    aggs, aggv = _edges(pair3, Dr, selTr, xr, idxc, idxr3, sW, dW, l13, params)
    st, xyz9 = _finish(h, aggs, aggv, l1, xr, params)
    return xyz9.reshape(1, L, 3, 3), st.reshape(1, L, L0_OUT)


# speedup vs baseline: 11.9311x; 1.1966x over previous
import functools

import jax
import jax.numpy as jnp
from jax import lax
from jax.experimental import pallas as pl
from jax.experimental.pallas import tpu as pltpu
from jax.experimental.pallas import tpu_sc as plsc

L = 512
TOPK = 128
BI = 8
NBLK = L // BI
D_MSA, D_PAIR, D_STATE = 256, 128, 16
L0_IN, L0_OUT, D_EDGE = 32, 16, 32
HID = 64


def _ln(x, g, b, eps=1e-5):
    mu = jnp.mean(x, axis=-1, keepdims=True)
    var = jnp.mean((x - mu) ** 2, axis=-1, keepdims=True)
    return (x - mu) * lax.rsqrt(var + eps) * g + b


def _dot(a, b):
    return jnp.dot(a, b, preferred_element_type=jnp.float32)


def _prep_body(msa_ref, st_ref, xr_ref, xT_ref,
               gmsa_ref, bmsa_ref, gst_ref, bst_ref,
               Wx_ref, bx_ref, gn_ref, bn_ref, W1_ref,
               h_ref, sW_ref, dW_ref, D_ref, sel_ref, l1_ref, pos_ref):
    m = _ln(msa_ref[...], gmsa_ref[...], bmsa_ref[...])
    st = _ln(st_ref[...], gst_ref[...], bst_ref[...])
    x = jnp.concatenate([m, st], axis=1)
    h = _ln(_dot(x, Wx_ref[...]) + bx_ref[...], gn_ref[...], bn_ref[...])
    h_ref[...] = h
    W1 = W1_ref[...]
    sW_ref[...] = _dot(h, W1[0:L0_IN, :])
    dW_ref[...] = _dot(h, W1[L0_IN:2 * L0_IN, :])

    xr = xr_ref[...]
    ca = xr[:, 3:6]
    l1_ref[...] = xr - jnp.concatenate([ca, ca, ca], axis=1)

    d2 = jnp.zeros((L, L), jnp.float32)
    for x3 in range(3):
        col = xr[:, 3 + x3:4 + x3]
        row = xT_ref[3 + x3:4 + x3, :]
        dx = col - row
        d2 = d2 + dx * dx
    D = jnp.sqrt(d2 + 1e-8)
    D_ref[...] = D

    ri = lax.broadcasted_iota(jnp.int32, (L, L), 0)
    ci = lax.broadcasted_iota(jnp.int32, (L, L), 1)
    Dg = D + jnp.where(ri == ci, jnp.float32(999.9), jnp.float32(0.0))
    bits = lax.bitcast_convert_type(Dg, jnp.int32)

    lo0 = jnp.zeros((L, 1), jnp.int32)
    hi0 = jnp.full((L, 1), jnp.int32(0x7F7FFFFF))

    def body(_, carry):
        lo, hi = carry
        mid = lo + (hi - lo) // 2
        cnt = jnp.sum((bits <= mid).astype(jnp.int32), axis=1, keepdims=True)
        ge = cnt >= TOPK
        return jnp.where(ge, lo, mid + 1), jnp.where(ge, mid, hi)

    lo, hi = lax.fori_loop(0, 31, body, (lo0, hi0))
    t = hi
    below = bits < t
    ties = bits == t
    cnt_lt = jnp.sum(below.astype(jnp.float32), axis=1, keepdims=True)
    need = jnp.float32(TOPK) - cnt_lt
    ties_f = ties.astype(jnp.float32)
    upper = (ri <= ci).astype(jnp.float32)
    rank = _dot(ties_f, upper)
    sel = jnp.where(below | (ties & (rank <= need)),
                    jnp.float32(1.0), jnp.float32(0.0))
    sel_ref[...] = sel
    posm = _dot(sel, upper)
    selb = sel > 0.5
    vali = jnp.where(selb, ci, jnp.int32(L))
    si = jnp.where(selb, ci - (posm.astype(jnp.int32) - 1), jnp.int32(0))
    for k in range(9):
        d = 1 << k
        mvi = si & d
        in_v = jnp.concatenate([vali[:, d:], vali[:, :d]], axis=1)
        in_s = jnp.concatenate([si[:, d:], si[:, :d]], axis=1)
        in_mi = jnp.concatenate([mvi[:, d:], mvi[:, :d]], axis=1)
        mv = mvi != 0
        in_m = in_mi != 0
        vali = jnp.where(in_m, in_v, jnp.where(mv, jnp.int32(L), vali))
        si = jnp.where(in_m, in_s - d, jnp.where(mv, jnp.int32(0), si))
    eidx = vali[:, :TOPK]
    pos_ref[...] = eidx + ri[:, :TOPK] * jnp.int32(L)


def _prep(msa0, state0, xr, xT, p):
    out_shapes = [
        jax.ShapeDtypeStruct((L, L0_IN), jnp.float32),
        jax.ShapeDtypeStruct((L, HID), jnp.float32),
        jax.ShapeDtypeStruct((L, HID), jnp.float32),
        jax.ShapeDtypeStruct((L, L), jnp.float32),
        jax.ShapeDtypeStruct((L, L), jnp.float32),
        jax.ShapeDtypeStruct((L, 9), jnp.float32),
        jax.ShapeDtypeStruct((L, TOPK), jnp.int32),
    ]
    r2 = lambda v: v.reshape(1, -1)
    args = (msa0, state0, xr, xT,
            r2(p['ln_msa'][0]), r2(p['ln_msa'][1]),
            r2(p['ln_state'][0]), r2(p['ln_state'][1]),
            p['embed_x'][0], r2(p['embed_x'][1]),
            r2(p['ln_node'][0]), r2(p['ln_node'][1]),
            p['se3_W1'][0])
    return pl.pallas_call(
        _prep_body,
        out_shape=out_shapes,
    )(*args)


TABW = 40


def _sc_gather(pidx_all, pair_flat):
    info = plsc.get_sparse_core_info()
    NC, NS = info.num_cores, info.num_subcores
    NW = NC * NS
    rows_per_w = L // NW

    mesh = plsc.VectorSubcoreMesh(core_axis_name="c", subcore_axis_name="s")

    @functools.partial(
        pl.kernel, mesh=mesh,
        out_type=jax.ShapeDtypeStruct((L * TOPK, D_PAIR), jnp.float32),
        scratch_types=[
            pltpu.VMEM((TOPK,), jnp.int32),
            pltpu.VMEM((TOPK, D_PAIR), jnp.float32),
            pltpu.VMEM((TOPK, D_PAIR), jnp.float32),
            pltpu.SemaphoreType.DMA,
            pltpu.SemaphoreType.DMA,
        ],
    )
    def k(pidx_hbm, pair_hbm, gpair_hbm, pidxv, gbuf0, gbuf1, sem0, sem1):
        wid = lax.axis_index("s") * NC + lax.axis_index("c")
        bufs = (gbuf0, gbuf1)
        sems = (sem0, sem1)
        for t in range(rows_per_w):
            row = wid * rows_per_w + t
            pltpu.sync_copy(pidx_hbm.at[row], pidxv)
            b = t & 1
            pltpu.async_copy(pair_hbm.at[pidxv], bufs[b], sems[b]).wait()
            pltpu.sync_copy(bufs[b], gpair_hbm.at[pl.ds(row * TOPK, TOPK)])

    return k(pidx_all, pair_flat)


def _edge_body(gpair_ref, q3_ref, tab_ref, xrb_ref, idxb_ref, sWb_ref, l13_ref,
               Wg1_ref, cg1_ref, cb1_ref, o128_ref,
               W2ag_ref, cg2_ref, We2r_ref, we2s_ref, cb2_ref, o32_ref,
               W1g_ref, cg3_ref, w1d_ref, cb3_ref, o64_ref, W2_ref, b2_ref,
               W1d_ref, agg_ref):
    @pl.when(pl.program_id(0) == 0)
    def _():
        agg_ref[...] = jnp.zeros_like(agg_ref)

    mu = (lax.broadcasted_iota(jnp.int32, (1, 36), 1).astype(jnp.float32)
          * jnp.float32(20.0 / 35.0))
    sig = jnp.float32(20.0 / 36)
    eps = jnp.float32(1e-5)
    iotai = lax.broadcasted_iota(jnp.int32, (TOPK, L), 1)
    agg_acc = jnp.zeros_like(agg_ref)
    i0 = pl.program_id(0) * BI
    for r in range(BI):
        base = r * TOPK
        x0 = gpair_ref[base:base + TOPK, :]
        jfi = q3_ref[0][:, r:r + 1] - (i0 + r) * jnp.int32(L)
        onehot = jnp.where(jfi == iotai, jnp.float32(1.0), jnp.float32(0.0))
        tabf = _dot(onehot, tab_ref[...])
        h_d = tabf[:, 0:L0_IN]
        ca_d = tabf[:, L0_IN:L0_IN + 3]
        idf = tabf[:, L0_IN + 3:L0_IN + 4]
        dWr = _dot(h_d, W1d_ref[...])
        diff = ca_d - xrb_ref[r:r + 1, 3:6]
        Dcol = jnp.sqrt(jnp.sum(diff * diff, axis=1, keepdims=True) + 1e-8)
        u3 = diff / (Dcol + 1e-6)
        dseq = idf - idxb_ref[0][0:1, r:r + 1]
        aseq = jnp.abs(dseq)
        seq = jnp.sign(dseq) * jnp.where(aseq > 1.0, 0.0, aseq)
        mu0 = _dot(x0, o128_ref[...])
        m20 = _dot(x0 * x0, o128_ref[...])
        rs0 = lax.rsqrt(m20 - mu0 * mu0 + eps)
        pre1 = rs0 * (_dot(x0, Wg1_ref[...]) - mu0 * cg1_ref[...]) + cb1_ref[...]
        mu1 = _dot(pre1, o32_ref[...])
        m21 = _dot(pre1 * pre1, o32_ref[...])
        rs1 = lax.rsqrt(m21 - mu1 * mu1 + eps)
        rbf = jnp.exp(-(((Dcol - mu) / sig) ** 2))
        pre2 = (rs1 * (_dot(pre1, W2ag_ref[...]) - mu1 * cg2_ref[...])
                + _dot(rbf, We2r_ref[...])
                + seq * we2s_ref[...] + cb2_ref[...])
        mu2 = _dot(pre2, o64_ref[...])
        m22 = _dot(pre2 * pre2, o64_ref[...])
        rs2 = lax.rsqrt(m22 - mu2 * mu2 + eps)
        hid = (rs2 * (_dot(pre2, W1g_ref[...]) - mu2 * cg3_ref[...])
               + Dcol * w1d_ref[...] + cb3_ref[...]
               + sWb_ref[r:r + 1, :] + dWr)
        hid = jnp.maximum(hid, 0.0)
        out = _dot(hid, W2_ref[...]) + b2_ref[...]
        l1m = l13_ref[r]
        mv0 = out[:, L0_IN:L0_IN + 1] * u3 + _dot(out[:, L0_IN + 1:L0_IN + 4], l1m)
        mv1 = out[:, L0_IN + 4:L0_IN + 5] * u3 + _dot(out[:, L0_IN + 5:L0_IN + 8], l1m)
        msgf = jnp.concatenate([out[:, :L0_IN], mv0, mv1], axis=1)
        agg_acc = agg_acc + lax.dot_general(
            onehot, msgf, (((0,), (0,)), ((), ())),
            preferred_element_type=jnp.float32)
    agg_ref[...] += agg_acc


def _edges(gpair, q3, tab, xr9b, idxr3, sW, l13, p):
    We2 = p['embed_e2'][0]
    W1 = p['se3_W1'][0]
    r2 = lambda v: v.reshape(1, -1)
    gp, bp = p['ln_pair']
    ge1, bn1 = p['ln_edge1']
    ge2, bn2 = p['ln_edge2']
    We1, be1 = p['embed_e1']
    We2a = We2[0:D_EDGE, :]
    We2r = We2[D_EDGE:D_EDGE + 36, :]
    we2s = We2[D_EDGE + 36:, :]
    W1pr = W1[2 * L0_IN:2 * L0_IN + D_EDGE, :]
    w1d = W1[2 * L0_IN + D_EDGE:, :]
    Wg1 = We1 * gp[:, None]
    cg1 = r2(jnp.sum(Wg1, axis=0))
    cb1 = r2(bp @ We1 + be1)
    W2ag = We2a * ge1[:, None]
    cg2 = r2(jnp.sum(W2ag, axis=0))
    cb2 = r2(bn1 @ We2a + p['embed_e2'][1])
    W1g = W1pr * ge2[:, None]
    cg3 = r2(jnp.sum(W1g, axis=0))
    cb3 = r2(bn2 @ W1pr + p['se3_W1'][1])
    o128 = jnp.full((D_PAIR, D_EDGE), 1.0 / D_PAIR, jnp.float32)
    o32 = jnp.full((D_EDGE, D_EDGE), 1.0 / D_EDGE, jnp.float32)
    o64 = jnp.full((D_EDGE, HID), 1.0 / D_EDGE, jnp.float32)
    W1d = W1[L0_IN:2 * L0_IN, :]
    args = (gpair, q3, tab, xr9b, idxr3, sW, l13,
            Wg1, cg1, cb1, o128,
            W2ag, cg2, We2r, we2s, cb2, o32,
            W1g, cg3, w1d, cb3, o64,
            p['se3_W2'][0], r2(p['se3_W2'][1]), W1d)
    in_specs = [
        pl.BlockSpec((BI * TOPK, D_PAIR), lambda i: (i, 0)),
        pl.BlockSpec((1, TOPK, BI), lambda i: (i, 0, 0)),
        pl.BlockSpec((L, TABW), lambda i: (0, 0)),
        pl.BlockSpec((BI, 9), lambda i: (i, 0)),
        pl.BlockSpec((1, 1, BI), lambda i: (i, 0, 0)),
        pl.BlockSpec((BI, HID), lambda i: (i, 0)),
        pl.BlockSpec((BI, 3, 3), lambda i: (i, 0, 0)),
    ] + [pl.BlockSpec(a.shape, functools.partial(lambda n, i: (0,) * n, a.ndim))
         for a in args[7:]]
    out_specs = pl.BlockSpec((L, L0_IN + 6), lambda i: (0, 0))
    out_shapes = jax.ShapeDtypeStruct((L, L0_IN + 6), jnp.float32)
    return pl.pallas_call(
        _edge_body,
        grid=(NBLK,),
        in_specs=in_specs,
        out_specs=out_specs,
        out_shape=out_shapes,
        compiler_params=pltpu.CompilerParams(
            dimension_semantics=("arbitrary",)),
    )(*args)


def _finish_body(h_ref, agg_ref, l1_ref, xr_ref,
                 W0_ref, b0_ref, st_ref, xyz_ref, eps=1e-5):
    W0 = W0_ref[...]
    st_ref[...] = (_dot(h_ref[...], W0[0:L0_IN, :])
                   + _dot(agg_ref[...][:, :L0_IN], W0[L0_IN:, :]) + b0_ref[...])
    aggv = agg_ref[...][:, L0_IN:L0_IN + 6] * jnp.float32(0.01)
    T = aggv[:, 0:3]
    R = aggv[:, 3:6]
    Ra = jnp.sqrt(jnp.sum(R * R, axis=1, keepdims=True) + 1e-12)
    Rv = R / (Ra + eps)
    cosA = jnp.cos(Ra)
    sinA = jnp.sin(Ra)
    ca = xr_ref[...][:, 3:6]
    l1 = l1_ref[...]
    shift = ca + T
    outs = []
    for b in range(3):
        v = l1[:, 3 * b:3 * b + 3]
        Rdv = jnp.sum(Rv * v, axis=1, keepdims=True)
        cr = jnp.concatenate([
            Rv[:, 1:2] * v[:, 2:3] - Rv[:, 2:3] * v[:, 1:2],
            Rv[:, 2:3] * v[:, 0:1] - Rv[:, 0:1] * v[:, 2:3],
            Rv[:, 0:1] * v[:, 1:2] - Rv[:, 1:2] * v[:, 0:1],
        ], axis=1)
        u_par = Rv * Rdv
        vnew = (v - u_par) * cosA + cr * sinA + u_par
        outs.append(vnew + shift)
    xyz_ref[...] = jnp.concatenate(outs, axis=1)


def _finish(h, agg, l1, xr, p):
    out_shapes = [
        jax.ShapeDtypeStruct((L, L0_OUT), jnp.float32),
        jax.ShapeDtypeStruct((L, 9), jnp.float32),
    ]
    return pl.pallas_call(
        _finish_body,
        out_shape=out_shapes,
    )(h, agg, l1, xr, p['se3_W0'][0], p['se3_W0'][1].reshape(1, -1))


def kernel(msa, pair, xyz, state, idx, top_k, params):
    del top_k
    msa0 = msa[0, 0]
    state0 = state[0]
    xr = xyz[0].reshape(L, 9).astype(jnp.float32)
    xT = xr.T
    pair3 = pair[0]
    idxc = idx[0].astype(jnp.float32).reshape(L, 1)

    h, sW, dW, D, sel, l1, pidx_all = _prep(msa0, state0, xr, xT, params)
    del dW, D, sel
    idxr3 = idxc.reshape(NBLK, 1, BI)
    l13 = l1.reshape(L, 3, 3)
    tab = jnp.concatenate(
        [h, xr[:, 3:6], idxc, jnp.zeros((L, 4), jnp.float32)], axis=1)
    gpair = _sc_gather(pidx_all, pair3.reshape(L * L, D_PAIR))
    q3 = pidx_all.reshape(NBLK, BI, TOPK).transpose(0, 2, 1)
    agg = _edges(gpair, q3, tab, xr, idxr3, sW, l13, params)
    st, xyz9 = _finish(h, agg, l1, xr, params)
    return xyz9.reshape(1, L, 3, 3), st.reshape(1, L, L0_OUT)


# speedup vs baseline: 12.1892x; 1.0216x over previous
import functools

import jax
import jax.numpy as jnp
from jax import lax
from jax.experimental import pallas as pl
from jax.experimental.pallas import tpu as pltpu
from jax.experimental.pallas import tpu_sc as plsc

L = 512
TOPK = 128
BI = 8
NBLK = L // BI
D_MSA, D_PAIR, D_STATE = 256, 128, 16
L0_IN, L0_OUT, D_EDGE = 32, 16, 32
HID = 64


def _ln(x, g, b, eps=1e-5):
    mu = jnp.mean(x, axis=-1, keepdims=True)
    var = jnp.mean((x - mu) ** 2, axis=-1, keepdims=True)
    return (x - mu) * lax.rsqrt(var + eps) * g + b


def _dot(a, b):
    return jnp.dot(a, b, preferred_element_type=jnp.float32)


def _prep_body(msa_ref, st_ref, xr_ref, xT_ref,
               gmsa_ref, bmsa_ref, gst_ref, bst_ref,
               Wx_ref, bx_ref, gn_ref, bn_ref, W1_ref,
               h_ref, sW_ref, dW_ref, D_ref, sel_ref, l1_ref, pos_ref):
    m = _ln(msa_ref[...], gmsa_ref[...], bmsa_ref[...])
    st = _ln(st_ref[...], gst_ref[...], bst_ref[...])
    x = jnp.concatenate([m, st], axis=1)
    h = _ln(_dot(x, Wx_ref[...]) + bx_ref[...], gn_ref[...], bn_ref[...])
    h_ref[...] = h
    W1 = W1_ref[...]
    sW_ref[...] = _dot(h, W1[0:L0_IN, :])
    dW_ref[...] = _dot(h, W1[L0_IN:2 * L0_IN, :])

    xr = xr_ref[...]
    ca = xr[:, 3:6]
    l1_ref[...] = xr - jnp.concatenate([ca, ca, ca], axis=1)

    d2 = jnp.zeros((L, L), jnp.float32)
    for x3 in range(3):
        col = xr[:, 3 + x3:4 + x3]
        row = xT_ref[3 + x3:4 + x3, :]
        dx = col - row
        d2 = d2 + dx * dx
    D = jnp.sqrt(d2 + 1e-8)
    D_ref[...] = D

    ri = lax.broadcasted_iota(jnp.int32, (L, L), 0)
    ci = lax.broadcasted_iota(jnp.int32, (L, L), 1)
    Dg = D + jnp.where(ri == ci, jnp.float32(999.9), jnp.float32(0.0))
    bits = lax.bitcast_convert_type(Dg, jnp.int32)

    lo0 = jnp.zeros((L, 1), jnp.int32)
    hi0 = jnp.full((L, 1), jnp.int32(0x7F7FFFFF))

    def body(_, carry):
        lo, hi = carry
        mid = lo + (hi - lo) // 2
        cnt = jnp.sum((bits <= mid).astype(jnp.int32), axis=1, keepdims=True)
        ge = cnt >= TOPK
        return jnp.where(ge, lo, mid + 1), jnp.where(ge, mid, hi)

    lo, hi = lax.fori_loop(0, 31, body, (lo0, hi0))
    t = hi
    below = bits < t
    ties = bits == t
    cnt_lt = jnp.sum(below.astype(jnp.float32), axis=1, keepdims=True)
    need = jnp.float32(TOPK) - cnt_lt
    ties_f = ties.astype(jnp.float32)
    upper = (ri <= ci).astype(jnp.float32)
    rank = _dot(ties_f, upper)
    sel = jnp.where(below | (ties & (rank <= need)),
                    jnp.float32(1.0), jnp.float32(0.0))
    sel_ref[...] = sel
    posm = _dot(sel, upper)
    selb = sel > 0.5
    vali = jnp.where(selb, ci, jnp.int32(L))
    si = jnp.where(selb, ci - (posm.astype(jnp.int32) - 1), jnp.int32(0))
    for k in range(9):
        d = 1 << k
        mvi = si & d
        in_v = jnp.concatenate([vali[:, d:], vali[:, :d]], axis=1)
        in_s = jnp.concatenate([si[:, d:], si[:, :d]], axis=1)
        in_mi = jnp.concatenate([mvi[:, d:], mvi[:, :d]], axis=1)
        mv = mvi != 0
        in_m = in_mi != 0
        vali = jnp.where(in_m, in_v, jnp.where(mv, jnp.int32(L), vali))
        si = jnp.where(in_m, in_s - d, jnp.where(mv, jnp.int32(0), si))
    eidx = vali[:, :TOPK]
    pos_ref[...] = eidx + ri[:, :TOPK] * jnp.int32(L)


def _prep(msa0, state0, xr, xT, p):
    out_shapes = [
        jax.ShapeDtypeStruct((L, L0_IN), jnp.float32),
        jax.ShapeDtypeStruct((L, HID), jnp.float32),
        jax.ShapeDtypeStruct((L, HID), jnp.float32),
        jax.ShapeDtypeStruct((L, L), jnp.float32),
        jax.ShapeDtypeStruct((L, L), jnp.float32),
        jax.ShapeDtypeStruct((L, 9), jnp.float32),
        jax.ShapeDtypeStruct((L, TOPK), jnp.int32),
    ]
    r2 = lambda v: v.reshape(1, -1)
    args = (msa0, state0, xr, xT,
            r2(p['ln_msa'][0]), r2(p['ln_msa'][1]),
            r2(p['ln_state'][0]), r2(p['ln_state'][1]),
            p['embed_x'][0], r2(p['embed_x'][1]),
            r2(p['ln_node'][0]), r2(p['ln_node'][1]),
            p['se3_W1'][0])
    return pl.pallas_call(
        _prep_body,
        out_shape=out_shapes,
    )(*args)


TABW = 40


def _sc_gather(pidx_all, pair_flat):
    info = plsc.get_sparse_core_info()
    NC, NS = info.num_cores, info.num_subcores
    NW = NC * NS
    rows_per_w = L // NW

    mesh = plsc.VectorSubcoreMesh(core_axis_name="c", subcore_axis_name="s")

    @functools.partial(
        pl.kernel, mesh=mesh,
        out_type=jax.ShapeDtypeStruct((L * TOPK, D_PAIR), jnp.float32),
        scratch_types=[
            pltpu.VMEM((TOPK,), jnp.int32),
            pltpu.VMEM((TOPK,), jnp.int32),
            pltpu.VMEM((TOPK, D_PAIR), jnp.float32),
            pltpu.VMEM((TOPK, D_PAIR), jnp.float32),
            pltpu.SemaphoreType.DMA,
            pltpu.SemaphoreType.DMA,
        ],
    )
    def k(pidx_hbm, pair_hbm, gpair_hbm, pidx0, pidx1, gbuf0, gbuf1,
          sem0, sem1):
        wid = lax.axis_index("s") * NC + lax.axis_index("c")
        base = wid * rows_per_w
        idxs = (pidx0, pidx1)
        bufs = (gbuf0, gbuf1)
        sems = (sem0, sem1)
        pltpu.sync_copy(pidx_hbm.at[base], pidx0)
        waits = [pltpu.async_copy(pair_hbm.at[pidx0], gbuf0, sem0)]
        for t in range(rows_per_w):
            if t + 1 < rows_per_w:
                b = (t + 1) & 1
                pltpu.sync_copy(pidx_hbm.at[base + t + 1], idxs[b])
                waits.append(
                    pltpu.async_copy(pair_hbm.at[idxs[b]], bufs[b], sems[b]))
            waits[t].wait()
            pltpu.sync_copy(bufs[t & 1],
                            gpair_hbm.at[pl.ds((base + t) * TOPK, TOPK)])

    return k(pidx_all, pair_flat)


def _edge_body(gpair_ref, q3_ref, tab_ref, xrb_ref, idxb_ref, sWb_ref, l13_ref,
               Wg1_ref, cg1_ref, cb1_ref, o128_ref,
               W2ag_ref, cg2_ref, We2r_ref, we2s_ref, cb2_ref, o32_ref,
               W1g_ref, cg3_ref, w1d_ref, cb3_ref, o64_ref, W2_ref, b2_ref,
               W1d_ref, agg_ref):
    @pl.when(pl.program_id(0) == 0)
    def _():
        agg_ref[...] = jnp.zeros_like(agg_ref)

    mu = (lax.broadcasted_iota(jnp.int32, (1, 36), 1).astype(jnp.float32)
          * jnp.float32(20.0 / 35.0))
    sig = jnp.float32(20.0 / 36)
    eps = jnp.float32(1e-5)
    iotai = lax.broadcasted_iota(jnp.int32, (TOPK, L), 1)
    agg_acc = jnp.zeros_like(agg_ref)
    i0 = pl.program_id(0) * BI
    for r in range(BI):
        base = r * TOPK
        x0 = gpair_ref[base:base + TOPK, :]
        jfi = q3_ref[0][:, r:r + 1] - (i0 + r) * jnp.int32(L)
        onehot = jnp.where(jfi == iotai, jnp.float32(1.0), jnp.float32(0.0))
        tabf = _dot(onehot, tab_ref[...])
        h_d = tabf[:, 0:L0_IN]
        ca_d = tabf[:, L0_IN:L0_IN + 3]
        idf = tabf[:, L0_IN + 3:L0_IN + 4]
        dWr = _dot(h_d, W1d_ref[...])
        diff = ca_d - xrb_ref[r:r + 1, 3:6]
        Dcol = jnp.sqrt(jnp.sum(diff * diff, axis=1, keepdims=True) + 1e-8)
        u3 = diff / (Dcol + 1e-6)
        dseq = idf - idxb_ref[0][0:1, r:r + 1]
        aseq = jnp.abs(dseq)
        seq = jnp.sign(dseq) * jnp.where(aseq > 1.0, 0.0, aseq)
        mu0 = _dot(x0, o128_ref[...])
        m20 = _dot(x0 * x0, o128_ref[...])
        rs0 = lax.rsqrt(m20 - mu0 * mu0 + eps)
        pre1 = rs0 * (_dot(x0, Wg1_ref[...]) - mu0 * cg1_ref[...]) + cb1_ref[...]
        mu1 = _dot(pre1, o32_ref[...])
        m21 = _dot(pre1 * pre1, o32_ref[...])
        rs1 = lax.rsqrt(m21 - mu1 * mu1 + eps)
        rbf = jnp.exp(-(((Dcol - mu) / sig) ** 2))
        pre2 = (rs1 * (_dot(pre1, W2ag_ref[...]) - mu1 * cg2_ref[...])
                + _dot(rbf, We2r_ref[...])
                + seq * we2s_ref[...] + cb2_ref[...])
        mu2 = _dot(pre2, o64_ref[...])
        m22 = _dot(pre2 * pre2, o64_ref[...])
        rs2 = lax.rsqrt(m22 - mu2 * mu2 + eps)
        hid = (rs2 * (_dot(pre2, W1g_ref[...]) - mu2 * cg3_ref[...])
               + Dcol * w1d_ref[...] + cb3_ref[...]
               + sWb_ref[r:r + 1, :] + dWr)
        hid = jnp.maximum(hid, 0.0)
        out = _dot(hid, W2_ref[...]) + b2_ref[...]
        l1m = l13_ref[r]
        mv0 = out[:, L0_IN:L0_IN + 1] * u3 + _dot(out[:, L0_IN + 1:L0_IN + 4], l1m)
        mv1 = out[:, L0_IN + 4:L0_IN + 5] * u3 + _dot(out[:, L0_IN + 5:L0_IN + 8], l1m)
        msgf = jnp.concatenate([out[:, :L0_IN], mv0, mv1], axis=1)
        agg_acc = agg_acc + lax.dot_general(
            onehot, msgf, (((0,), (0,)), ((), ())),
            preferred_element_type=jnp.float32)
    agg_ref[...] += agg_acc


def _edges(gpair, q3, tab, xr9b, idxr3, sW, l13, p):
    We2 = p['embed_e2'][0]
    W1 = p['se3_W1'][0]
    r2 = lambda v: v.reshape(1, -1)
    gp, bp = p['ln_pair']
    ge1, bn1 = p['ln_edge1']
    ge2, bn2 = p['ln_edge2']
    We1, be1 = p['embed_e1']
    We2a = We2[0:D_EDGE, :]
    We2r = We2[D_EDGE:D_EDGE + 36, :]
    we2s = We2[D_EDGE + 36:, :]
    W1pr = W1[2 * L0_IN:2 * L0_IN + D_EDGE, :]
    w1d = W1[2 * L0_IN + D_EDGE:, :]
    Wg1 = We1 * gp[:, None]
    cg1 = r2(jnp.sum(Wg1, axis=0))
    cb1 = r2(bp @ We1 + be1)
    W2ag = We2a * ge1[:, None]
    cg2 = r2(jnp.sum(W2ag, axis=0))
    cb2 = r2(bn1 @ We2a + p['embed_e2'][1])
    W1g = W1pr * ge2[:, None]
    cg3 = r2(jnp.sum(W1g, axis=0))
    cb3 = r2(bn2 @ W1pr + p['se3_W1'][1])
    o128 = jnp.full((D_PAIR, D_EDGE), 1.0 / D_PAIR, jnp.float32)
    o32 = jnp.full((D_EDGE, D_EDGE), 1.0 / D_EDGE, jnp.float32)
    o64 = jnp.full((D_EDGE, HID), 1.0 / D_EDGE, jnp.float32)
    W1d = W1[L0_IN:2 * L0_IN, :]
    args = (gpair, q3, tab, xr9b, idxr3, sW, l13,
            Wg1, cg1, cb1, o128,
            W2ag, cg2, We2r, we2s, cb2, o32,
            W1g, cg3, w1d, cb3, o64,
            p['se3_W2'][0], r2(p['se3_W2'][1]), W1d)
    in_specs = [
        pl.BlockSpec((BI * TOPK, D_PAIR), lambda i: (i, 0)),
        pl.BlockSpec((1, TOPK, BI), lambda i: (i, 0, 0)),
        pl.BlockSpec((L, TABW), lambda i: (0, 0)),
        pl.BlockSpec((BI, 9), lambda i: (i, 0)),
        pl.BlockSpec((1, 1, BI), lambda i: (i, 0, 0)),
        pl.BlockSpec((BI, HID), lambda i: (i, 0)),
        pl.BlockSpec((BI, 3, 3), lambda i: (i, 0, 0)),
    ] + [pl.BlockSpec(a.shape, functools.partial(lambda n, i: (0,) * n, a.ndim))
         for a in args[7:]]
    out_specs = pl.BlockSpec((L, L0_IN + 6), lambda i: (0, 0))
    out_shapes = jax.ShapeDtypeStruct((L, L0_IN + 6), jnp.float32)
    return pl.pallas_call(
        _edge_body,
        grid=(NBLK,),
        in_specs=in_specs,
        out_specs=out_specs,
        out_shape=out_shapes,
        compiler_params=pltpu.CompilerParams(
            dimension_semantics=("arbitrary",)),
    )(*args)


def _finish_body(h_ref, agg_ref, l1_ref, xr_ref,
                 W0_ref, b0_ref, st_ref, xyz_ref, eps=1e-5):
    W0 = W0_ref[...]
    st_ref[...] = (_dot(h_ref[...], W0[0:L0_IN, :])
                   + _dot(agg_ref[...][:, :L0_IN], W0[L0_IN:, :]) + b0_ref[...])
    aggv = agg_ref[...][:, L0_IN:L0_IN + 6] * jnp.float32(0.01)
    T = aggv[:, 0:3]
    R = aggv[:, 3:6]
    Ra = jnp.sqrt(jnp.sum(R * R, axis=1, keepdims=True) + 1e-12)
    Rv = R / (Ra + eps)
    cosA = jnp.cos(Ra)
    sinA = jnp.sin(Ra)
    ca = xr_ref[...][:, 3:6]
    l1 = l1_ref[...]
    shift = ca + T
    outs = []
    for b in range(3):
        v = l1[:, 3 * b:3 * b + 3]
        Rdv = jnp.sum(Rv * v, axis=1, keepdims=True)
        cr = jnp.concatenate([
            Rv[:, 1:2] * v[:, 2:3] - Rv[:, 2:3] * v[:, 1:2],
            Rv[:, 2:3] * v[:, 0:1] - Rv[:, 0:1] * v[:, 2:3],
            Rv[:, 0:1] * v[:, 1:2] - Rv[:, 1:2] * v[:, 0:1],
        ], axis=1)
        u_par = Rv * Rdv
        vnew = (v - u_par) * cosA + cr * sinA + u_par
        outs.append(vnew + shift)
    xyz_ref[...] = jnp.concatenate(outs, axis=1)


def _finish(h, agg, l1, xr, p):
    out_shapes = [
        jax.ShapeDtypeStruct((L, L0_OUT), jnp.float32),
        jax.ShapeDtypeStruct((L, 9), jnp.float32),
    ]
    return pl.pallas_call(
        _finish_body,
        out_shape=out_shapes,
    )(h, agg, l1, xr, p['se3_W0'][0], p['se3_W0'][1].reshape(1, -1))


def kernel(msa, pair, xyz, state, idx, top_k, params):
    del top_k
    msa0 = msa[0, 0]
    state0 = state[0]
    xr = xyz[0].reshape(L, 9).astype(jnp.float32)
    xT = xr.T
    pair3 = pair[0]
    idxc = idx[0].astype(jnp.float32).reshape(L, 1)

    h, sW, dW, D, sel, l1, pidx_all = _prep(msa0, state0, xr, xT, params)
    del dW, D, sel
    idxr3 = idxc.reshape(NBLK, 1, BI)
    l13 = l1.reshape(L, 3, 3)
    tab = jnp.concatenate(
        [h, xr[:, 3:6], idxc, jnp.zeros((L, 4), jnp.float32)], axis=1)
    gpair = _sc_gather(pidx_all, pair3.reshape(L * L, D_PAIR))
    q3 = pidx_all.reshape(NBLK, BI, TOPK).transpose(0, 2, 1)
    agg = _edges(gpair, q3, tab, xr, idxr3, sW, l13, params)
    st, xyz9 = _finish(h, agg, l1, xr, params)
    return xyz9.reshape(1, L, 3, 3), st.reshape(1, L, L0_OUT)
